# baseline (speedup 1.0000x reference)
import functools
import math

import jax
import jax.numpy as jnp
from jax import lax
from jax.experimental import pallas as pl
from jax.experimental.pallas import tpu as pltpu
from jax.experimental.pallas import tpu_sc as plsc

_VOCAB = 1000000
_D = 64
_SCALE = math.sqrt(_D)

_B = 4096
_T = 200
_NC = 2
_NS = 16
_NW = _NC * _NS
_RPW = _B // _NW
_K = 2
_NSUP = _RPW // _K

_mesh = plsc.VectorSubcoreMesh(core_axis_name="c", subcore_axis_name="s")


@functools.partial(
    pl.kernel,
    mesh=_mesh,
    out_type=jax.ShapeDtypeStruct((_B, _T, _D), jnp.float32),
    scratch_types=[
        pltpu.VMEM((_RPW, _T), jnp.int32),
        pltpu.VMEM((_K, _T, _D), jnp.float32),
        pltpu.VMEM((_K, _T, _D), jnp.float32),
        pltpu.SemaphoreType.DMA((_K,)),
        pltpu.SemaphoreType.DMA((_K,)),
    ],
    compiler_params=pltpu.CompilerParams(use_tc_tiling_on_sc=False),
)
def _embed_sc(idx_hbm, table_hbm, out_hbm, idx_v, gbuf, sbuf, gsem, ssem):
    wid = lax.axis_index("s") * _NC + lax.axis_index("c")
    r0 = wid * _RPW
    pltpu.sync_copy(idx_hbm.at[pl.ds(r0, _RPW)], idx_v)

    def gather_start(i, b):
        pltpu.make_async_copy(
            table_hbm.at[idx_v.at[i]], gbuf.at[b], gsem.at[b]
        ).start()

    def gather_wait(i, b):
        pltpu.make_async_copy(
            table_hbm.at[idx_v.at[i]], gbuf.at[b], gsem.at[b]
        ).wait()

    def store_start(i, b):
        pltpu.make_async_copy(
            sbuf.at[b], out_hbm.at[r0 + i], ssem.at[b]
        ).start()

    def store_wait(i, b):
        pltpu.make_async_copy(
            sbuf.at[b], out_hbm.at[r0 + i], ssem.at[b]
        ).wait()

    for b in range(_K):
        gather_start(b, b)

    def superstep(s, carry):
        for b in range(_K):
            i = s * _K + b
            gather_wait(i, b)

            @pl.when(s > 0)
            def _():
                store_wait(i - _K, b)

            def row_body(r, c2):
                for c in range(_D // 16):
                    sl = pl.ds(c * 16, 16)
                    sbuf[b, r, sl] = gbuf[b, r, sl] * _SCALE
                return c2

            lax.fori_loop(0, _T, row_body, 0, unroll=4)

            @pl.when(s < _NSUP - 1)
            def _():
                gather_start(i + _K, b)

            store_start(i, b)
        return carry

    lax.fori_loop(0, _NSUP, superstep, 0)

    for b in range(_K):
        store_wait((_NSUP - 1) * _K + b, b)


def kernel(input_word, embed_weight):
    return _embed_sc(input_word.astype(jnp.int32), embed_weight)
